# Initial kernel scaffold; baseline (speedup 1.0000x reference)
import jax
import jax.numpy as jnp
from jax.experimental import pallas as pl


def kernel(x, edge_index, batch, W1a, b1a, W2a, b2a, W1b, b1b, W2b, b2b):
    raise NotImplementedError("write your pallas kernel here")


# speedup vs baseline: 10.7296x; 10.7296x over previous
import functools

import jax
import jax.numpy as jnp
from jax import lax
from jax.experimental import pallas as pl
from jax.experimental.pallas import tpu as pltpu
from jax.experimental.pallas import tpu_sc as plsc

N, E, D, H = 10000, 320000, 128, 128
NC, NS = 2, 16
NW = NC * NS
E_PER_W = E // NW
CHUNK = 80
NCH = E_PER_W // CHUNK
NG = 5
GCH = NCH // NG
RPT = 624
TAIL = N - NS * RPT

_mesh = plsc.VectorSubcoreMesh(core_axis_name="c", subcore_axis_name="s")


@functools.partial(
    pl.kernel,
    out_type=jax.ShapeDtypeStruct((NC, N, D), jnp.float32),
    mesh=_mesh,
    scratch_types=[
        pltpu.VMEM((GCH, CHUNK), jnp.int32),
        pltpu.VMEM((GCH, CHUNK), jnp.int32),
        pltpu.VMEM((2, CHUNK, D), jnp.float32),
        pltpu.VMEM_SHARED((N, D), jnp.float32),
        pltpu.SemaphoreType.DMA,
        pltpu.SemaphoreType.DMA,
    ],
)
def _agg(h_hbm, src_hbm, dst_hbm, out_hbm, src_v, dst_v, rows_v, acc_sh, sem0, sem1):
    cid = lax.axis_index("c")
    sid = lax.axis_index("s")
    wid = sid * NC + cid

    pltpu.sync_copy(h_hbm.at[pl.ds(sid * RPT, RPT)], acc_sh.at[pl.ds(sid * RPT, RPT)])

    @pl.when(sid == 0)
    def _():
        pltpu.sync_copy(h_hbm.at[pl.ds(NS * RPT, TAIL)], acc_sh.at[pl.ds(NS * RPT, TAIL)])
    plsc.subcore_barrier()

    def group(g, _):
        pltpu.sync_copy(src_hbm.at[wid, g], src_v)
        pltpu.sync_copy(dst_hbm.at[wid, g], dst_v)

        pltpu.async_copy(h_hbm.at[src_v.at[0]], rows_v.at[0], sem0)

        def pair(p, _):
            j0 = 2 * p
            pltpu.async_copy(h_hbm.at[src_v.at[j0 + 1]], rows_v.at[1], sem1)
            pltpu.make_async_copy(h_hbm.at[src_v.at[j0]], rows_v.at[0], sem0).wait()
            pltpu.sync_copy(rows_v.at[0], acc_sh.at[dst_v.at[j0]], add=True)

            @pl.when(j0 + 2 < GCH)
            def _():
                pltpu.async_copy(h_hbm.at[src_v.at[j0 + 2]], rows_v.at[0], sem0)

            pltpu.make_async_copy(h_hbm.at[src_v.at[j0 + 1]], rows_v.at[1], sem1).wait()
            pltpu.sync_copy(rows_v.at[1], acc_sh.at[dst_v.at[j0 + 1]], add=True)
            return 0

        lax.fori_loop(0, GCH // 2, pair, 0)
        pltpu.make_async_copy(h_hbm.at[src_v.at[GCH - 1]], rows_v.at[0], sem0).wait()
        pltpu.sync_copy(rows_v.at[0], acc_sh.at[dst_v.at[GCH - 1]], add=True)
        return 0

    lax.fori_loop(0, NG, group, 0)

    plsc.subcore_barrier()
    pltpu.sync_copy(acc_sh.at[pl.ds(sid * RPT, RPT)],
                    out_hbm.at[cid, pl.ds(sid * RPT, RPT)])

    @pl.when(sid == 0)
    def _():
        pltpu.sync_copy(acc_sh.at[pl.ds(NS * RPT, TAIL)],
                        out_hbm.at[cid, pl.ds(NS * RPT, TAIL)])


BLK = 2000


def _mlp_body(h_ref, p0_ref, p1_ref, w1_ref, b1_ref, w2_ref, b2_ref, o_ref):
    m = p0_ref[...] + p1_ref[...] - h_ref[...]
    t = jnp.dot(m, w1_ref[...], preferred_element_type=jnp.float32) + b1_ref[...]
    t = jnp.maximum(t, 0.0)
    o = jnp.dot(t, w2_ref[...], preferred_element_type=jnp.float32) + b2_ref[...]
    o_ref[...] = jnp.maximum(o, 0.0)


def _mlp(h, p0, p1, W1, b1, W2, b2):
    row_spec = pl.BlockSpec((BLK, D), lambda i: (i, 0))
    full = pl.BlockSpec((D, H), lambda i: (0, 0))
    bias = pl.BlockSpec((1, H), lambda i: (0, 0))
    return pl.pallas_call(
        _mlp_body,
        grid=(N // BLK,),
        in_specs=[row_spec, row_spec, row_spec, full, bias, full, bias],
        out_specs=pl.BlockSpec((BLK, H), lambda i: (i, 0)),
        out_shape=jax.ShapeDtypeStruct((N, H), jnp.float32),
    )(h, p0, p1, W1, b1.reshape(1, H), W2, b2.reshape(1, H))


def kernel(x, edge_index, batch, W1a, b1a, W2a, b2a, W1b, b1b, W2b, b2b):
    src3 = edge_index[0].reshape(NW, NG, GCH, CHUNK)
    dst3 = edge_index[1].reshape(NW, NG, GCH, CHUNK)
    p = _agg(x, src3, dst3)
    h1 = _mlp(x, p[0], p[1], W1a, b1a, W2a, b2a)
    p2 = _agg(h1, src3, dst3)
    return _mlp(h1, p2[0], p2[1], W1b, b1b, W2b, b2b)


# speedup vs baseline: 12.1369x; 1.1312x over previous
import functools

import jax
import jax.numpy as jnp
from jax import lax
from jax.experimental import pallas as pl
from jax.experimental.pallas import tpu as pltpu
from jax.experimental.pallas import tpu_sc as plsc

N, E, D, H = 10000, 320000, 128, 128
NC, NS = 2, 16
NW = NC * NS
E_PER_W = E // NW
CHUNK = 80
NCH = E_PER_W // CHUNK
NG = 5
GCH = NCH // NG
RPT = 624
TAIL = N - NS * RPT

_mesh = plsc.VectorSubcoreMesh(core_axis_name="c", subcore_axis_name="s")


@functools.partial(
    pl.kernel,
    out_type=jax.ShapeDtypeStruct((NC, N, D), jnp.float32),
    mesh=_mesh,
    scratch_types=[
        pltpu.VMEM((GCH, CHUNK), jnp.int32),
        pltpu.VMEM((GCH, CHUNK), jnp.int32),
        pltpu.VMEM((3, CHUNK, D), jnp.float32),
        pltpu.VMEM_SHARED((N, D), jnp.float32),
        pltpu.SemaphoreType.DMA,
        pltpu.SemaphoreType.DMA,
        pltpu.SemaphoreType.DMA,
        pltpu.SemaphoreType.DMA,
        pltpu.SemaphoreType.DMA,
        pltpu.SemaphoreType.DMA,
    ],
)
def _agg(h_hbm, src_hbm, dst_hbm, out_hbm, src_v, dst_v, rows_v, acc_sh,
         g0, g1, g2, s0, s1, s2):
    cid = lax.axis_index("c")
    sid = lax.axis_index("s")
    wid = sid * NC + cid

    pltpu.sync_copy(h_hbm.at[pl.ds(sid * RPT, RPT)], acc_sh.at[pl.ds(sid * RPT, RPT)])

    @pl.when(sid == 0)
    def _():
        pltpu.sync_copy(h_hbm.at[pl.ds(NS * RPT, TAIL)], acc_sh.at[pl.ds(NS * RPT, TAIL)])
    plsc.subcore_barrier()

    gsem = (g0, g1, g2)
    ssem = (s0, s1, s2)

    def _gath(j, s):
        pltpu.async_copy(h_hbm.at[src_v.at[j]], rows_v.at[s], gsem[s])

    def _wait_g(j, s):
        pltpu.make_async_copy(h_hbm.at[src_v.at[j]], rows_v.at[s], gsem[s]).wait()

    def _scat(j, s):
        pltpu.async_copy(rows_v.at[s], acc_sh.at[dst_v.at[j]], ssem[s], add=True)

    def _wait_s(j, s):
        pltpu.make_async_copy(rows_v.at[s], acc_sh.at[dst_v.at[j]], ssem[s]).wait()

    def group(g, _):
        pltpu.sync_copy(src_hbm.at[wid, g], src_v)
        pltpu.sync_copy(dst_hbm.at[wid, g], dst_v)

        _gath(0, 0)
        _gath(1, 1)

        def triple(q, _):
            j0 = 3 * q
            @pl.when(q > 0)
            def _():
                _wait_s(j0 - 1, 2)
            _gath(j0 + 2, 2)
            _wait_g(j0, 0)
            _scat(j0, 0)
            _wait_s(j0, 0)
            _gath(j0 + 3, 0)
            _wait_g(j0 + 1, 1)
            _scat(j0 + 1, 1)
            _wait_s(j0 + 1, 1)

            @pl.when(j0 + 4 < GCH)
            def _():
                _gath(j0 + 4, 1)
            _wait_g(j0 + 2, 2)
            _scat(j0 + 2, 2)
            return 0

        lax.fori_loop(0, GCH // 3, triple, 0)
        _wait_s(GCH - 2, 2)
        _wait_g(GCH - 1, 0)
        _scat(GCH - 1, 0)
        _wait_s(GCH - 1, 0)
        return 0

    lax.fori_loop(0, NG, group, 0)

    plsc.subcore_barrier()
    pltpu.sync_copy(acc_sh.at[pl.ds(sid * RPT, RPT)],
                    out_hbm.at[cid, pl.ds(sid * RPT, RPT)])

    @pl.when(sid == 0)
    def _():
        pltpu.sync_copy(acc_sh.at[pl.ds(NS * RPT, TAIL)],
                        out_hbm.at[cid, pl.ds(NS * RPT, TAIL)])


BLK = 2000


def _mlp_body(h_ref, p0_ref, p1_ref, w1_ref, b1_ref, w2_ref, b2_ref, o_ref):
    m = p0_ref[...] + p1_ref[...] - h_ref[...]
    t = jnp.dot(m, w1_ref[...], preferred_element_type=jnp.float32) + b1_ref[...]
    t = jnp.maximum(t, 0.0)
    o = jnp.dot(t, w2_ref[...], preferred_element_type=jnp.float32) + b2_ref[...]
    o_ref[...] = jnp.maximum(o, 0.0)


def _mlp(h, p0, p1, W1, b1, W2, b2):
    row_spec = pl.BlockSpec((BLK, D), lambda i: (i, 0))
    full = pl.BlockSpec((D, H), lambda i: (0, 0))
    bias = pl.BlockSpec((1, H), lambda i: (0, 0))
    return pl.pallas_call(
        _mlp_body,
        grid=(N // BLK,),
        in_specs=[row_spec, row_spec, row_spec, full, bias, full, bias],
        out_specs=pl.BlockSpec((BLK, H), lambda i: (i, 0)),
        out_shape=jax.ShapeDtypeStruct((N, H), jnp.float32),
    )(h, p0, p1, W1, b1.reshape(1, H), W2, b2.reshape(1, H))


def kernel(x, edge_index, batch, W1a, b1a, W2a, b2a, W1b, b1b, W2b, b2b):
    src3 = edge_index[0].reshape(NW, NG, GCH, CHUNK)
    dst3 = edge_index[1].reshape(NW, NG, GCH, CHUNK)
    p = _agg(x, src3, dst3)
    h1 = _mlp(x, p[0], p[1], W1a, b1a, W2a, b2a)
    p2 = _agg(h1, src3, dst3)
    return _mlp(h1, p2[0], p2[1], W1b, b1b, W2b, b2b)


# speedup vs baseline: 12.6769x; 1.0445x over previous
import functools

import jax
import jax.numpy as jnp
from jax import lax
from jax.experimental import pallas as pl
from jax.experimental.pallas import tpu as pltpu
from jax.experimental.pallas import tpu_sc as plsc

N, E, D, H = 10000, 320000, 128, 128
NC, NS = 2, 16
NW = NC * NS
E_PER_W = E // NW
CHUNK = 80
NCH = E_PER_W // CHUNK
NG = 5
GCH = NCH // NG
RPT = 624
TAIL = N - NS * RPT

_mesh = plsc.VectorSubcoreMesh(core_axis_name="c", subcore_axis_name="s")


@functools.partial(
    pl.kernel,
    out_type=jax.ShapeDtypeStruct((NC, N, D), jnp.float32),
    mesh=_mesh,
    scratch_types=[
        pltpu.VMEM((2, GCH, CHUNK), jnp.int32),
        pltpu.VMEM((2, GCH, CHUNK), jnp.int32),
        pltpu.VMEM((3, CHUNK, D), jnp.float32),
        pltpu.VMEM_SHARED((N, D), jnp.float32),
        pltpu.SemaphoreType.DMA,
        pltpu.SemaphoreType.DMA,
        pltpu.SemaphoreType.DMA,
        pltpu.SemaphoreType.DMA,
        pltpu.SemaphoreType.DMA,
        pltpu.SemaphoreType.DMA,
        pltpu.SemaphoreType.DMA,
    ],
)
def _agg(h_hbm, src_hbm, dst_hbm, out_hbm, src_v, dst_v, rows_v, acc_sh,
         g0, g1, g2, s0, s1, s2, isem):
    cid = lax.axis_index("c")
    sid = lax.axis_index("s")
    wid = sid * NC + cid

    pltpu.sync_copy(h_hbm.at[pl.ds(sid * RPT, RPT)], acc_sh.at[pl.ds(sid * RPT, RPT)])

    @pl.when(sid == 0)
    def _():
        pltpu.sync_copy(h_hbm.at[pl.ds(NS * RPT, TAIL)], acc_sh.at[pl.ds(NS * RPT, TAIL)])
    plsc.subcore_barrier()

    gsem = (g0, g1, g2)
    ssem = (s0, s1, s2)

    def _stage_idx(g, slot):
        pltpu.async_copy(src_hbm.at[wid, g], src_v.at[slot], isem)
        pltpu.async_copy(dst_hbm.at[wid, g], dst_v.at[slot], isem)

    def _wait_idx(g, slot):
        pltpu.make_async_copy(src_hbm.at[wid, g], src_v.at[slot], isem).wait()
        pltpu.make_async_copy(dst_hbm.at[wid, g], dst_v.at[slot], isem).wait()

    _stage_idx(0, 0)

    def group(g, _):
        slot = lax.rem(g, 2)

        def _gath(j, s):
            pltpu.async_copy(h_hbm.at[src_v.at[slot, j]], rows_v.at[s], gsem[s])

        def _wait_g(j, s):
            pltpu.make_async_copy(h_hbm.at[src_v.at[slot, j]], rows_v.at[s], gsem[s]).wait()

        def _scat(j, s):
            pltpu.async_copy(rows_v.at[s], acc_sh.at[dst_v.at[slot, j]], ssem[s], add=True)

        def _wait_s(j, s):
            pltpu.make_async_copy(rows_v.at[s], acc_sh.at[dst_v.at[slot, j]], ssem[s]).wait()

        _wait_idx(g, slot)

        @pl.when(g + 1 < NG)
        def _():
            _stage_idx(g + 1, 1 - slot)

        _gath(0, 0)
        _gath(1, 1)

        def triple(q, _):
            j0 = 3 * q
            @pl.when(q > 0)
            def _():
                _wait_s(j0 - 1, 2)
            _gath(j0 + 2, 2)
            _wait_g(j0, 0)
            _scat(j0, 0)
            _wait_s(j0, 0)
            _gath(j0 + 3, 0)
            _wait_g(j0 + 1, 1)
            _scat(j0 + 1, 1)
            _wait_s(j0 + 1, 1)

            @pl.when(j0 + 4 < GCH)
            def _():
                _gath(j0 + 4, 1)
            _wait_g(j0 + 2, 2)
            _scat(j0 + 2, 2)
            return 0

        lax.fori_loop(0, GCH // 3, triple, 0)
        _wait_s(GCH - 2, 2)
        _wait_g(GCH - 1, 0)
        _scat(GCH - 1, 0)
        _wait_s(GCH - 1, 0)
        return 0

    lax.fori_loop(0, NG, group, 0)

    plsc.subcore_barrier()
    pltpu.sync_copy(acc_sh.at[pl.ds(sid * RPT, RPT)],
                    out_hbm.at[cid, pl.ds(sid * RPT, RPT)])

    @pl.when(sid == 0)
    def _():
        pltpu.sync_copy(acc_sh.at[pl.ds(NS * RPT, TAIL)],
                        out_hbm.at[cid, pl.ds(NS * RPT, TAIL)])


BLK = 2000


def _mlp_body(h_ref, p0_ref, p1_ref, w1_ref, b1_ref, w2_ref, b2_ref, o_ref):
    m = p0_ref[...] + p1_ref[...] - h_ref[...]
    t = jnp.dot(m, w1_ref[...], preferred_element_type=jnp.float32) + b1_ref[...]
    t = jnp.maximum(t, 0.0)
    o = jnp.dot(t, w2_ref[...], preferred_element_type=jnp.float32) + b2_ref[...]
    o_ref[...] = jnp.maximum(o, 0.0)


def _mlp(h, p0, p1, W1, b1, W2, b2):
    row_spec = pl.BlockSpec((BLK, D), lambda i: (i, 0))
    full = pl.BlockSpec((D, H), lambda i: (0, 0))
    bias = pl.BlockSpec((1, H), lambda i: (0, 0))
    return pl.pallas_call(
        _mlp_body,
        grid=(N // BLK,),
        in_specs=[row_spec, row_spec, row_spec, full, bias, full, bias],
        out_specs=pl.BlockSpec((BLK, H), lambda i: (i, 0)),
        out_shape=jax.ShapeDtypeStruct((N, H), jnp.float32),
    )(h, p0, p1, W1, b1.reshape(1, H), W2, b2.reshape(1, H))


def kernel(x, edge_index, batch, W1a, b1a, W2a, b2a, W1b, b1b, W2b, b2b):
    src3 = edge_index[0].reshape(NW, NG, GCH, CHUNK)
    dst3 = edge_index[1].reshape(NW, NG, GCH, CHUNK)
    p = _agg(x, src3, dst3)
    h1 = _mlp(x, p[0], p[1], W1a, b1a, W2a, b2a)
    p2 = _agg(h1, src3, dst3)
    return _mlp(h1, p2[0], p2[1], W1b, b1b, W2b, b2b)


# speedup vs baseline: 13.3473x; 1.0529x over previous
import functools

import jax
import jax.numpy as jnp
from jax import lax
from jax.experimental import pallas as pl
from jax.experimental.pallas import tpu as pltpu
from jax.experimental.pallas import tpu_sc as plsc

N, E, D, H = 10000, 320000, 128, 128
NC, NS = 2, 16
NW = NC * NS
E_PER_W = E // NW
CHUNK = 80
NCH = E_PER_W // CHUNK
NG = 5
GCH = NCH // NG
RPT = 624
TAIL = N - NS * RPT

_mesh = plsc.VectorSubcoreMesh(core_axis_name="c", subcore_axis_name="s")


@functools.partial(
    pl.kernel,
    out_type=jax.ShapeDtypeStruct((NC, N, D), jnp.float32),
    mesh=_mesh,
    scratch_types=[
        pltpu.VMEM((2, GCH, CHUNK), jnp.int32),
        pltpu.VMEM((2, GCH, CHUNK), jnp.int32),
        pltpu.VMEM((3, CHUNK, D), jnp.float32),
        pltpu.VMEM_SHARED((N, D), jnp.float32),
        pltpu.SemaphoreType.DMA,
        pltpu.SemaphoreType.DMA,
        pltpu.SemaphoreType.DMA,
        pltpu.SemaphoreType.DMA,
        pltpu.SemaphoreType.DMA,
        pltpu.SemaphoreType.DMA,
        pltpu.SemaphoreType.DMA,
    ],
)
def _agg(h_hbm, src_hbm, dst_hbm, out_hbm, src_v, dst_v, rows_v, acc_sh,
         g0, g1, g2, s0, s1, s2, isem):
    cid = lax.axis_index("c")
    sid = lax.axis_index("s")
    wid = sid * NC + cid

    pltpu.sync_copy(h_hbm.at[pl.ds(sid * RPT, RPT)], acc_sh.at[pl.ds(sid * RPT, RPT)])

    @pl.when(sid == 0)
    def _():
        pltpu.sync_copy(h_hbm.at[pl.ds(NS * RPT, TAIL)], acc_sh.at[pl.ds(NS * RPT, TAIL)])
    plsc.subcore_barrier()

    gsem = (g0, g1, g2)
    ssem = (s0, s1, s2)

    def _stage_idx(g, slot):
        pltpu.async_copy(src_hbm.at[wid, g], src_v.at[slot], isem)
        pltpu.async_copy(dst_hbm.at[wid, g], dst_v.at[slot], isem)

    def _wait_idx(g, slot):
        pltpu.make_async_copy(src_hbm.at[wid, g], src_v.at[slot], isem).wait()
        pltpu.make_async_copy(dst_hbm.at[wid, g], dst_v.at[slot], isem).wait()

    _stage_idx(0, 0)

    def group(g, _):
        slot = lax.rem(g, 2)

        def _gath(j, s):
            pltpu.async_copy(h_hbm.at[src_v.at[slot, j]], rows_v.at[s], gsem[s])

        def _wait_g(j, s):
            pltpu.make_async_copy(h_hbm.at[src_v.at[slot, j]], rows_v.at[s], gsem[s]).wait()

        def _scat(j, s):
            pltpu.async_copy(rows_v.at[s], acc_sh.at[dst_v.at[slot, j]], ssem[s], add=True)

        def _wait_s(j, s):
            pltpu.make_async_copy(rows_v.at[s], acc_sh.at[dst_v.at[slot, j]], ssem[s]).wait()

        _wait_idx(g, slot)

        @pl.when(g + 1 < NG)
        def _():
            _stage_idx(g + 1, 1 - slot)

        _gath(0, 0)
        _gath(1, 1)

        def triple(q, _):
            j0 = 3 * q
            @pl.when(q > 0)
            def _():
                _wait_s(j0 - 1, 2)
            _gath(j0 + 2, 2)
            _wait_g(j0, 0)
            _scat(j0, 0)
            _wait_s(j0, 0)
            _gath(j0 + 3, 0)
            _wait_g(j0 + 1, 1)
            _scat(j0 + 1, 1)
            _wait_s(j0 + 1, 1)

            @pl.when(j0 + 4 < GCH)
            def _():
                _gath(j0 + 4, 1)
            _wait_g(j0 + 2, 2)
            _scat(j0 + 2, 2)
            return 0

        lax.fori_loop(0, GCH // 3, triple, 0)
        _wait_s(GCH - 2, 2)
        _wait_g(GCH - 1, 0)
        _scat(GCH - 1, 0)
        _wait_s(GCH - 1, 0)
        return 0

    lax.fori_loop(0, NG, group, 0)

    plsc.subcore_barrier()
    pltpu.sync_copy(acc_sh.at[pl.ds(sid * RPT, RPT)],
                    out_hbm.at[cid, pl.ds(sid * RPT, RPT)])

    @pl.when(sid == 0)
    def _():
        pltpu.sync_copy(acc_sh.at[pl.ds(NS * RPT, TAIL)],
                        out_hbm.at[cid, pl.ds(NS * RPT, TAIL)])


BLK = 2000


def _mlp_body(h_ref, p0_ref, p1_ref, w1_ref, b1_ref, w2_ref, b2_ref, o_ref):
    m = p0_ref[0] + p1_ref[0] - h_ref[...]
    t = jnp.dot(m, w1_ref[...], preferred_element_type=jnp.float32) + b1_ref[...]
    t = jnp.maximum(t, 0.0)
    o = jnp.dot(t, w2_ref[...], preferred_element_type=jnp.float32) + b2_ref[...]
    o_ref[...] = jnp.maximum(o, 0.0)


def _mlp(h, p, W1, b1, W2, b2):
    row_spec = pl.BlockSpec((BLK, D), lambda i: (i, 0))
    full = pl.BlockSpec((D, H), lambda i: (0, 0))
    bias = pl.BlockSpec((1, H), lambda i: (0, 0))
    return pl.pallas_call(
        _mlp_body,
        grid=(N // BLK,),
        in_specs=[row_spec,
                  pl.BlockSpec((1, BLK, D), lambda i: (0, i, 0)),
                  pl.BlockSpec((1, BLK, D), lambda i: (1, i, 0)),
                  full, bias, full, bias],
        out_specs=pl.BlockSpec((BLK, H), lambda i: (i, 0)),
        out_shape=jax.ShapeDtypeStruct((N, H), jnp.float32),
    )(h, p, p, W1, b1.reshape(1, H), W2, b2.reshape(1, H))


def kernel(x, edge_index, batch, W1a, b1a, W2a, b2a, W1b, b1b, W2b, b2b):
    src3 = edge_index[0].reshape(NW, NG, GCH, CHUNK)
    dst3 = edge_index[1].reshape(NW, NG, GCH, CHUNK)
    p = _agg(x, src3, dst3)
    h1 = _mlp(x, p, W1a, b1a, W2a, b2a)
    p2 = _agg(h1, src3, dst3)
    return _mlp(h1, p2, W1b, b1b, W2b, b2b)


# speedup vs baseline: 14.0406x; 1.0519x over previous
import functools

import jax
import jax.numpy as jnp
from jax import lax
from jax.experimental import pallas as pl
from jax.experimental.pallas import tpu as pltpu
from jax.experimental.pallas import tpu_sc as plsc

N, E, D, H = 10000, 320000, 128, 128
NC, NS = 2, 16
NW = NC * NS
E_PER_W = E // NW
CHUNK = 80
NCH = E_PER_W // CHUNK
NG = 5
GCH = NCH // NG
RPT = 624
TAIL = N - NS * RPT

_mesh = plsc.VectorSubcoreMesh(core_axis_name="c", subcore_axis_name="s")


@functools.partial(
    pl.kernel,
    out_type=jax.ShapeDtypeStruct((NC, N, D), jnp.float32),
    mesh=_mesh,
    scratch_types=[
        pltpu.VMEM((2, GCH, CHUNK), jnp.int32),
        pltpu.VMEM((2, GCH, CHUNK), jnp.int32),
        pltpu.VMEM((3, CHUNK, D), jnp.float32),
        pltpu.VMEM_SHARED((N, D), jnp.float32),
        pltpu.SemaphoreType.DMA,
        pltpu.SemaphoreType.DMA,
        pltpu.SemaphoreType.DMA,
        pltpu.SemaphoreType.DMA,
        pltpu.SemaphoreType.DMA,
        pltpu.SemaphoreType.DMA,
        pltpu.SemaphoreType.DMA,
    ],
)
def _agg(h_hbm, ei_hbm, out_hbm, src_v, dst_v, rows_v, acc_sh,
         g0, g1, g2, s0, s1, s2, isem):
    cid = lax.axis_index("c")
    sid = lax.axis_index("s")
    wid = sid * NC + cid

    gsem = (g0, g1, g2)
    ssem = (s0, s1, s2)

    def _stage_idx(g, slot):
        pltpu.async_copy(ei_hbm.at[0, wid, g], src_v.at[slot], isem)
        pltpu.async_copy(ei_hbm.at[1, wid, g], dst_v.at[slot], isem)

    def _wait_idx(g, slot):
        pltpu.make_async_copy(ei_hbm.at[0, wid, g], src_v.at[slot], isem).wait()
        pltpu.make_async_copy(ei_hbm.at[1, wid, g], dst_v.at[slot], isem).wait()

    _stage_idx(0, 0)

    pltpu.sync_copy(h_hbm.at[pl.ds(sid * RPT, RPT)], acc_sh.at[pl.ds(sid * RPT, RPT)])

    @pl.when(sid == 0)
    def _():
        pltpu.sync_copy(h_hbm.at[pl.ds(NS * RPT, TAIL)], acc_sh.at[pl.ds(NS * RPT, TAIL)])
    plsc.subcore_barrier()

    def group(g, _):
        slot = lax.rem(g, 2)

        def _gath(j, s):
            pltpu.async_copy(h_hbm.at[src_v.at[slot, j]], rows_v.at[s], gsem[s])

        def _wait_g(j, s):
            pltpu.make_async_copy(h_hbm.at[src_v.at[slot, j]], rows_v.at[s], gsem[s]).wait()

        def _scat(j, s):
            pltpu.async_copy(rows_v.at[s], acc_sh.at[dst_v.at[slot, j]], ssem[s], add=True)

        def _wait_s(j, s):
            pltpu.make_async_copy(rows_v.at[s], acc_sh.at[dst_v.at[slot, j]], ssem[s]).wait()

        _wait_idx(g, slot)

        @pl.when(g + 1 < NG)
        def _():
            _stage_idx(g + 1, 1 - slot)

        _gath(0, 0)
        _gath(1, 1)

        def triple(q, _):
            j0 = 3 * q
            @pl.when(q > 0)
            def _():
                _wait_s(j0 - 1, 2)
            _gath(j0 + 2, 2)
            _wait_g(j0, 0)
            _scat(j0, 0)
            _wait_s(j0, 0)
            _gath(j0 + 3, 0)
            _wait_g(j0 + 1, 1)
            _scat(j0 + 1, 1)
            _wait_s(j0 + 1, 1)

            @pl.when(j0 + 4 < GCH)
            def _():
                _gath(j0 + 4, 1)
            _wait_g(j0 + 2, 2)
            _scat(j0 + 2, 2)
            return 0

        lax.fori_loop(0, GCH // 3, triple, 0)
        _wait_s(GCH - 2, 2)
        _wait_g(GCH - 1, 0)
        _scat(GCH - 1, 0)
        _wait_s(GCH - 1, 0)
        return 0

    lax.fori_loop(0, NG, group, 0)

    plsc.subcore_barrier()
    pltpu.sync_copy(acc_sh.at[pl.ds(sid * RPT, RPT)],
                    out_hbm.at[cid, pl.ds(sid * RPT, RPT)])

    @pl.when(sid == 0)
    def _():
        pltpu.sync_copy(acc_sh.at[pl.ds(NS * RPT, TAIL)],
                        out_hbm.at[cid, pl.ds(NS * RPT, TAIL)])


BLK = 2000


def _mlp_body(h_ref, p0_ref, p1_ref, w1_ref, b1_ref, w2_ref, b2_ref, o_ref):
    m = p0_ref[0] + p1_ref[0] - h_ref[...]
    t = jnp.dot(m, w1_ref[...], preferred_element_type=jnp.float32) + b1_ref[...]
    t = jnp.maximum(t, 0.0)
    o = jnp.dot(t, w2_ref[...], preferred_element_type=jnp.float32) + b2_ref[...]
    o_ref[...] = jnp.maximum(o, 0.0)


def _mlp(h, p, W1, b1, W2, b2):
    row_spec = pl.BlockSpec((BLK, D), lambda i: (i, 0))
    full = pl.BlockSpec((D, H), lambda i: (0, 0))
    bias = pl.BlockSpec((1, H), lambda i: (0, 0))
    return pl.pallas_call(
        _mlp_body,
        grid=(N // BLK,),
        in_specs=[row_spec,
                  pl.BlockSpec((1, BLK, D), lambda i: (0, i, 0)),
                  pl.BlockSpec((1, BLK, D), lambda i: (1, i, 0)),
                  full, bias, full, bias],
        out_specs=pl.BlockSpec((BLK, H), lambda i: (i, 0)),
        out_shape=jax.ShapeDtypeStruct((N, H), jnp.float32),
    )(h, p, p, W1, b1.reshape(1, H), W2, b2.reshape(1, H))


def kernel(x, edge_index, batch, W1a, b1a, W2a, b2a, W1b, b1b, W2b, b2b):
    ei5 = edge_index.reshape(2, NW, NG, GCH, CHUNK)
    p = _agg(x, ei5)
    h1 = _mlp(x, p, W1a, b1a, W2a, b2a)
    p2 = _agg(h1, ei5)
    return _mlp(h1, p2, W1b, b1b, W2b, b2b)
